# baseline (speedup 1.0000x reference)
import functools
import math

import jax
import jax.numpy as jnp
from jax import lax
from jax.experimental import pallas as pl
from jax.experimental.pallas import tpu as pltpu
from jax.experimental.pallas import tpu_sc as plsc

_NC = 2
_NS = 16
_NW = _NC * _NS

_D = 768
_GROUPS = _D // 16
_K = 32
_RING = 3


def _body(ids_hbm, table_hbm, pe_hbm, out_hbm, idx_v, pe0,
          rows0, rows1, rows2, sg0, sg1, sg2, ss0, ss1, ss2, sp0, sp1):
    batch, seq_len = ids_hbm.shape
    pos_per_w = seq_len // _NW
    n_chunks = pos_per_w // _K
    n_tasks = n_chunks * batch
    scale = jnp.float32(math.sqrt(float(_D)))

    rows = [rows0, rows1, rows2]
    sems_g = [sg0, sg1, sg2]
    sems_s = [ss0, ss1, ss2]

    w = lax.axis_index("s") * _NC + lax.axis_index("c")
    w_base = w * pos_per_w

    def issue_gather(t):
        pc, b = divmod(t, batch)
        return pltpu.async_copy(
            table_hbm.at[idx_v.at[b, pl.ds(pc * _K, _K)]],
            rows[t % _RING], sems_g[t % _RING])

    def issue_store(t):
        pc, b = divmod(t, batch)
        return pltpu.async_copy(
            rows[t % _RING],
            out_hbm.at[b, pl.ds(w_base + pc * _K, _K), :],
            sems_s[t % _RING])

    def compute(t):
        pc = t // batch
        rv = rows[t % _RING]
        off = pc * _K

        def row_fma(i, carry):
            for j in range(_GROUPS):
                sl = pl.ds(j * 16, 16)
                rv[i, sl] = rv[i, sl] * scale + pe0[off + i, sl]
            return carry
        lax.fori_loop(0, _K, row_fma, 0)

    idx_cps = [
        pltpu.async_copy(ids_hbm.at[b, pl.ds(w_base, pos_per_w)],
                         idx_v.at[b], sp1)
        for b in range(batch)
    ]
    for cp in idx_cps:
        cp.wait()

    pe_cp = pltpu.async_copy(
        pe_hbm.at[0, pl.ds(w_base, pos_per_w), :], pe0, sp0)
    gathers = {0: issue_gather(0)}
    stores = {}

    pe_cp.wait()
    for t in range(n_tasks):
        if t + 1 < n_tasks:
            if t - 2 >= 0:
                stores[t - 2].wait()
            gathers[t + 1] = issue_gather(t + 1)
        gathers[t].wait()
        compute(t)
        stores[t] = issue_store(t)
    for t in range(n_tasks - _RING, n_tasks):
        stores[t].wait()


def kernel(input_ids, word_embeddings, pe):
    batch, seq_len = input_ids.shape
    ids32 = input_ids.astype(jnp.int32)
    pos_per_w = seq_len // _NW

    mesh = plsc.VectorSubcoreMesh(
        core_axis_name="c", subcore_axis_name="s",
        num_cores=_NC, num_subcores=_NS,
    )
    run = pl.kernel(
        _body,
        out_type=jax.ShapeDtypeStruct((batch, seq_len, _D), jnp.float32),
        mesh=mesh,
        scratch_types=[
            pltpu.VMEM((batch, pos_per_w), jnp.int32),
            pltpu.VMEM((pos_per_w, _D), jnp.float32),
            pltpu.VMEM((_K, _D), jnp.float32),
            pltpu.VMEM((_K, _D), jnp.float32),
            pltpu.VMEM((_K, _D), jnp.float32),
            pltpu.SemaphoreType.DMA,
            pltpu.SemaphoreType.DMA,
            pltpu.SemaphoreType.DMA,
            pltpu.SemaphoreType.DMA,
            pltpu.SemaphoreType.DMA,
            pltpu.SemaphoreType.DMA,
            pltpu.SemaphoreType.DMA,
            pltpu.SemaphoreType.DMA,
        ],
    )
    return run(ids32, word_embeddings, pe)


# speedup vs baseline: 1.3543x; 1.3543x over previous
import functools
import math

import jax
import jax.numpy as jnp
from jax import lax
from jax.experimental import pallas as pl
from jax.experimental.pallas import tpu as pltpu
from jax.experimental.pallas import tpu_sc as plsc

_NC = 2
_NS = 16
_NW = _NC * _NS

_D = 768
_GROUPS = _D // 16
_K = 32
_RING = 3


def _body(ids_hbm, table_hbm, pe_hbm, out_hbm, idx_v, pe0, pe1,
          rows0, rows1, rows2, sg0, sg1, sg2, ss0, ss1, ss2, sp0, sp1):
    batch, seq_len = ids_hbm.shape
    pos_per_w = seq_len // _NW
    n_chunks = pos_per_w // _K
    n_tasks = n_chunks * batch
    scale = jnp.float32(math.sqrt(float(_D)))

    rows = [rows0, rows1, rows2]
    sems_g = [sg0, sg1, sg2]
    sems_s = [ss0, ss1, ss2]

    w = lax.axis_index("s") * _NC + lax.axis_index("c")
    w_base = w * pos_per_w

    def issue_gather(t):
        pc, b = divmod(t, batch)
        return pltpu.async_copy(
            table_hbm.at[idx_v.at[b, pl.ds(pc * _K, _K)]],
            rows[t % _RING], sems_g[t % _RING])

    def issue_store(t):
        pc, b = divmod(t, batch)
        return pltpu.async_copy(
            rows[t % _RING],
            out_hbm.at[b, pl.ds(w_base + pc * _K, _K), :],
            sems_s[t % _RING])

    def compute(t):
        pc = t // batch
        rv = rows[t % _RING]
        pv = pe0 if pc == 0 else pe1

        def row_fma(i, carry):
            for j in range(_GROUPS):
                sl = pl.ds(j * 16, 16)
                rv[i, sl] = rv[i, sl] * scale + pv[i, sl]
            return carry
        lax.fori_loop(0, _K, row_fma, 0)

    idx_cps = [
        pltpu.async_copy(ids_hbm.at[b, pl.ds(w_base, pos_per_w)],
                         idx_v.at[b], sp1)
        for b in range(batch)
    ]
    for cp in idx_cps:
        cp.wait()

    pe_cps = [
        pltpu.async_copy(pe_hbm.at[0, pl.ds(w_base, _K), :], pe0, sp0),
        pltpu.async_copy(pe_hbm.at[0, pl.ds(w_base + _K, _K), :], pe1, sp0),
    ]
    gathers = {0: issue_gather(0)}
    stores = {}

    for cp in pe_cps:
        cp.wait()
    for t in range(n_tasks):
        if t + 1 < n_tasks:
            if t - 2 >= 0:
                stores[t - 2].wait()
            gathers[t + 1] = issue_gather(t + 1)
        gathers[t].wait()
        compute(t)
        stores[t] = issue_store(t)
    for t in range(n_tasks - _RING, n_tasks):
        stores[t].wait()


def kernel(input_ids, word_embeddings, pe):
    batch, seq_len = input_ids.shape
    ids32 = input_ids.astype(jnp.int32)
    pos_per_w = seq_len // _NW

    mesh = plsc.VectorSubcoreMesh(
        core_axis_name="c", subcore_axis_name="s",
        num_cores=_NC, num_subcores=_NS,
    )
    run = pl.kernel(
        _body,
        out_type=jax.ShapeDtypeStruct((batch, seq_len, _D), jnp.float32),
        mesh=mesh,
        scratch_types=[
            pltpu.VMEM((batch, pos_per_w), jnp.int32),
            pltpu.VMEM((_K, _D), jnp.float32),
            pltpu.VMEM((_K, _D), jnp.float32),
            pltpu.VMEM((_K, _D), jnp.float32),
            pltpu.VMEM((_K, _D), jnp.float32),
            pltpu.VMEM((_K, _D), jnp.float32),
            pltpu.SemaphoreType.DMA,
            pltpu.SemaphoreType.DMA,
            pltpu.SemaphoreType.DMA,
            pltpu.SemaphoreType.DMA,
            pltpu.SemaphoreType.DMA,
            pltpu.SemaphoreType.DMA,
            pltpu.SemaphoreType.DMA,
            pltpu.SemaphoreType.DMA,
        ],
    )
    return run(ids32, word_embeddings, pe)
